# Initial kernel scaffold; baseline (speedup 1.0000x reference)
import jax
import jax.numpy as jnp
from jax.experimental import pallas as pl


def kernel(node_solu, edge_solu, ei_solu, bt_solu, node_solv1, edge_solv1, ei_solv1, bt_solv1, node_solv2, edge_solv2, ei_solv2, bt_solv2, g_solv_facs, temp, gsolu_pW, gsolu_pb, gsolu_e1W, gsolu_e1b, gsolu_e2W, gsolu_e2b, gsolu_nnb, gsolu_Wih, gsolu_Whh, gsolu_bih, gsolu_bhh, gsolv_pW, gsolv_pb, gsolv_e1W, gsolv_e1b, gsolv_e2W, gsolv_e2b, gsolv_nnb, gsolv_Wih, gsolv_Whh, gsolv_bih, gsolv_bhh, s2s_Wih0, s2s_Whh0, s2s_bih0, s2s_bhh0, s2s_Wih1, s2s_Whh1, s2s_bih1, s2s_bhh1, fc1W, fc1b, fc2W, fc2b):
    raise NotImplementedError("write your pallas kernel here")


# speedup vs baseline: 2.3060x; 2.3060x over previous
import functools

import jax
import jax.numpy as jnp
from jax import lax
from jax.experimental import pallas as pl
from jax.experimental.pallas import tpu as pltpu
from jax.experimental.pallas import tpu_sc as plsc

_N = 2560
_E = 5120
_B = 64
_D = 64
_EHID = 128
_NODE_IN = 128
_EDGE_IN = 16
_NT = 3
_NTOT = _NT * _N
_ETOT = _NT * _E
_NMP = 3
_NS2S = 3

_NC = 2
_NS = 16
_NW = _NC * _NS
_EPW = _ETOT // _NW
_CH = 120
_NCH = _EPW // _CH
_RPS = _NTOT // _NS

_HP = 128
_EB = 512
_NBLK = _E // _EB
_ICHUNK = 8


@functools.lru_cache(maxsize=None)
def _sc_kernels():
    mesh = plsc.VectorSubcoreMesh(core_axis_name="c", subcore_axis_name="s")

    @functools.partial(
        pl.kernel,
        mesh=mesh,
        out_type=jax.ShapeDtypeStruct((_ETOT, _HP), jnp.float32),
        scratch_types=[
            pltpu.VMEM((_CH,), jnp.int32),
            pltpu.VMEM((_CH, _HP), jnp.float32),
            pltpu.SemaphoreType.DMA,
        ],
    )
    def _gather_rows(h_hbm, src_hbm, out_hbm, idx_v, rows_v, sem):
        wid = lax.axis_index("c") * _NS + lax.axis_index("s")
        base = wid * _EPW
        for j in range(_NCH):
            off = base + j * _CH
            pltpu.sync_copy(src_hbm.at[pl.ds(off, _CH)], idx_v)
            pltpu.async_copy(h_hbm.at[idx_v], rows_v, sem).wait()
            pltpu.sync_copy(rows_v, out_hbm.at[pl.ds(off, _CH)])

    @functools.partial(
        pl.kernel,
        mesh=mesh,
        out_type=jax.ShapeDtypeStruct((_NC * _NTOT, _HP), jnp.float32),
        scratch_types=[
            pltpu.VMEM((_NCH, _CH), jnp.int32),
            pltpu.VMEM((_CH, _HP), jnp.float32),
            pltpu.VMEM_SHARED((_NTOT, _HP), jnp.float32),
        ],
    )
    def _scatter_add(m_hbm, dst_hbm, zero_hbm, out_hbm, idx_v, m_v, agg_sh):
        cid = lax.axis_index("c")
        sid = lax.axis_index("s")
        wid = cid * _NS + sid
        pltpu.sync_copy(zero_hbm, agg_sh.at[pl.ds(sid * _RPS, _RPS)])
        plsc.subcore_barrier()
        pltpu.sync_copy(dst_hbm.at[wid], idx_v)
        for j in range(_NCH):
            pltpu.sync_copy(m_hbm.at[pl.ds(wid * _EPW + j * _CH, _CH)], m_v)
            pltpu.sync_copy(m_v, agg_sh.at[idx_v.at[j]], add=True)
        plsc.subcore_barrier()
        pltpu.sync_copy(
            agg_sh.at[pl.ds(sid * _RPS, _RPS)],
            out_hbm.at[pl.ds(cid * _NTOT + sid * _RPS, _RPS)],
        )

    return _gather_rows, _scatter_add


def _proj_body(nf_ref, ef_ref, pw_ref, pb_ref, e1w_ref, e1b_ref, h_ref, z_ref):
    h = jnp.dot(nf_ref[...], pw_ref[0], preferred_element_type=jnp.float32)
    h = jnp.maximum(h + pb_ref[0], 0.0)
    h_ref[...] = jnp.concatenate([h, jnp.zeros((_N, _HP - _D), jnp.float32)], 1)
    z = jnp.dot(ef_ref[...], e1w_ref[0], preferred_element_type=jnp.float32)
    z_ref[...] = jnp.maximum(z + e1b_ref[0], 0.0)


_proj = pl.pallas_call(
    _proj_body,
    grid=(_NT,),
    in_specs=[
        pl.BlockSpec((_N, _NODE_IN), lambda t: (t, 0)),
        pl.BlockSpec((_E, _EDGE_IN), lambda t: (t, 0)),
        pl.BlockSpec((1, _NODE_IN, _D), lambda t: (t, 0, 0)),
        pl.BlockSpec((1, 1, _D), lambda t: (t, 0, 0)),
        pl.BlockSpec((1, _EDGE_IN, _EHID), lambda t: (t, 0, 0)),
        pl.BlockSpec((1, 1, _EHID), lambda t: (t, 0, 0)),
    ],
    out_specs=[
        pl.BlockSpec((_N, _HP), lambda t: (t, 0)),
        pl.BlockSpec((_E, _EHID), lambda t: (t, 0)),
    ],
    out_shape=[
        jax.ShapeDtypeStruct((_NTOT, _HP), jnp.float32),
        jax.ShapeDtypeStruct((_ETOT, _EHID), jnp.float32),
    ],
)


def _message_body(z_ref, hs_ref, w2_ref, b2_ref, m_ref):
    z = z_ref[...]
    hs = hs_ref[...]
    acc = jnp.zeros((_EB, _D), jnp.float32)
    for i0 in range(0, _D, _ICHUNK):
        w2c = w2_ref[0][:, i0 * _D:(i0 + _ICHUNK) * _D]
        ewc = jnp.dot(z, w2c, preferred_element_type=jnp.float32)
        ewc = ewc + b2_ref[0][:, i0 * _D:(i0 + _ICHUNK) * _D]
        for i in range(_ICHUNK):
            acc = acc + hs[:, i0 + i:i0 + i + 1] * ewc[:, i * _D:(i + 1) * _D]
    m_ref[...] = jnp.concatenate(
        [acc, jnp.zeros((_EB, _HP - _D), jnp.float32)], 1)


_message = pl.pallas_call(
    _message_body,
    grid=(_NT, _NBLK),
    in_specs=[
        pl.BlockSpec((_EB, _EHID), lambda t, b: (t * _NBLK + b, 0)),
        pl.BlockSpec((_EB, _HP), lambda t, b: (t * _NBLK + b, 0)),
        pl.BlockSpec((1, _EHID, _D * _D), lambda t, b: (t, 0, 0)),
        pl.BlockSpec((1, 1, _D * _D), lambda t, b: (t, 0, 0)),
    ],
    out_specs=pl.BlockSpec((_EB, _HP), lambda t, b: (t * _NBLK + b, 0)),
    out_shape=jax.ShapeDtypeStruct((_ETOT, _HP), jnp.float32),
)


def _gru_body(a2_ref, h_ref, nnb_ref, wih_ref, whh_ref, bih_ref, bhh_ref,
              hout_ref):
    agg = a2_ref[0][:, :_D] + a2_ref[1][:, :_D] + nnb_ref[0]
    x = jnp.maximum(agg, 0.0)
    h = h_ref[...][:, :_D]
    gi = jnp.dot(x, wih_ref[0], preferred_element_type=jnp.float32) + bih_ref[0]
    gh = jnp.dot(h, whh_ref[0], preferred_element_type=jnp.float32) + bhh_ref[0]
    r = jax.nn.sigmoid(gi[:, :_D] + gh[:, :_D])
    z = jax.nn.sigmoid(gi[:, _D:2 * _D] + gh[:, _D:2 * _D])
    n = jnp.tanh(gi[:, 2 * _D:] + r * gh[:, 2 * _D:])
    hnew = (1.0 - z) * n + z * h
    hout_ref[...] = jnp.concatenate(
        [hnew, jnp.zeros((_N, _HP - _D), jnp.float32)], 1)


_gru = pl.pallas_call(
    _gru_body,
    grid=(_NT,),
    in_specs=[
        pl.BlockSpec((2, _N, _HP), lambda t: (0, t, 0)),
        pl.BlockSpec((_N, _HP), lambda t: (t, 0)),
        pl.BlockSpec((1, 1, _D), lambda t: (t, 0, 0)),
        pl.BlockSpec((1, _D, 3 * _D), lambda t: (t, 0, 0)),
        pl.BlockSpec((1, _D, 3 * _D), lambda t: (t, 0, 0)),
        pl.BlockSpec((1, 1, 3 * _D), lambda t: (t, 0, 0)),
        pl.BlockSpec((1, 1, 3 * _D), lambda t: (t, 0, 0)),
    ],
    out_specs=pl.BlockSpec((_N, _HP), lambda t: (t, 0)),
    out_shape=jax.ShapeDtypeStruct((_NTOT, _HP), jnp.float32),
)


def _head_body(h_ref, bt_ref, wih0_ref, whh0_ref, bih0_ref, bhh0_ref,
               wih1_ref, whh1_ref, bih1_ref, bhh1_ref, facs_ref, temp_ref,
               fc1wa_ref, fc1wb_ref, fc1b_ref, fc2w_ref, fc2b_ref, out_ref):
    f32 = jnp.float32
    h_all = h_ref[...]
    bt_all = bt_ref[...]
    iota_b = lax.broadcasted_iota(jnp.int32, (_B, _N), 0)
    q_stars = []
    for t in range(_NT):
        feat = h_all[t * _N:(t + 1) * _N, :_D]
        feat_t = feat.T
        seg = jnp.broadcast_to(bt_all[t:t + 1, :], (_B, _N))
        mask = seg == iota_b
        h0 = jnp.zeros((_B, _D), f32)
        c0 = jnp.zeros((_B, _D), f32)
        h1 = jnp.zeros((_B, _D), f32)
        c1 = jnp.zeros((_B, _D), f32)
        q_star = jnp.zeros((_B, 2 * _D), f32)
        for _ in range(_NS2S):
            g0 = (jnp.dot(q_star, wih0_ref[...], preferred_element_type=f32)
                  + bih0_ref[0]
                  + jnp.dot(h0, whh0_ref[...], preferred_element_type=f32)
                  + bhh0_ref[0])
            c0 = (jax.nn.sigmoid(g0[:, _D:2 * _D]) * c0
                  + jax.nn.sigmoid(g0[:, :_D]) * jnp.tanh(g0[:, 2 * _D:3 * _D]))
            h0 = jax.nn.sigmoid(g0[:, 3 * _D:]) * jnp.tanh(c0)
            g1 = (jnp.dot(h0, wih1_ref[...], preferred_element_type=f32)
                  + bih1_ref[0]
                  + jnp.dot(h1, whh1_ref[...], preferred_element_type=f32)
                  + bhh1_ref[0])
            c1 = (jax.nn.sigmoid(g1[:, _D:2 * _D]) * c1
                  + jax.nn.sigmoid(g1[:, :_D]) * jnp.tanh(g1[:, 2 * _D:3 * _D]))
            h1 = jax.nn.sigmoid(g1[:, 3 * _D:]) * jnp.tanh(c1)
            q = h1
            s = jnp.dot(q, feat_t, preferred_element_type=f32)
            smask = jnp.where(mask, s, -jnp.inf)
            emax = jnp.max(smask, axis=1, keepdims=True)
            emax = jnp.where(emax > -3e38, emax, 0.0)
            a = jnp.where(mask, jnp.exp(s - emax), 0.0)
            denom = jnp.sum(a, axis=1, keepdims=True)
            denom = jnp.where(denom == 0.0, 1.0, denom)
            readout = jnp.dot(a / denom, feat, preferred_element_type=f32)
            q_star = jnp.concatenate([q, readout], axis=1)
        q_stars.append(q_star)
    facs = facs_ref[...]
    gb = facs[:, 0:1] * q_stars[1] + facs[:, 1:2] * q_stars[2]
    tnorm = (temp_ref[...] - 30.0) / 15.0
    x = jnp.concatenate([q_stars[0], gb], axis=1)
    hid = (jnp.dot(x, fc1wa_ref[...], preferred_element_type=f32)
           + tnorm * fc1wb_ref[...] + fc1b_ref[...])
    hid = jnp.maximum(hid, 0.0)
    out_ref[...] = (jnp.dot(hid, fc2w_ref[...], preferred_element_type=f32)
                    + fc2b_ref[...])


_head = pl.pallas_call(
    _head_body,
    out_shape=jax.ShapeDtypeStruct((_B, 1), jnp.float32),
)


def kernel(node_solu, edge_solu, ei_solu, bt_solu,
           node_solv1, edge_solv1, ei_solv1, bt_solv1,
           node_solv2, edge_solv2, ei_solv2, bt_solv2,
           g_solv_facs, temp,
           gsolu_pW, gsolu_pb, gsolu_e1W, gsolu_e1b, gsolu_e2W, gsolu_e2b,
           gsolu_nnb, gsolu_Wih, gsolu_Whh, gsolu_bih, gsolu_bhh,
           gsolv_pW, gsolv_pb, gsolv_e1W, gsolv_e1b, gsolv_e2W, gsolv_e2b,
           gsolv_nnb, gsolv_Wih, gsolv_Whh, gsolv_bih, gsolv_bhh,
           s2s_Wih0, s2s_Whh0, s2s_bih0, s2s_bhh0,
           s2s_Wih1, s2s_Whh1, s2s_bih1, s2s_bhh1,
           fc1W, fc1b, fc2W, fc2b):
    nf = jnp.concatenate([node_solu, node_solv1, node_solv2], 0)
    ef = jnp.concatenate([edge_solu, edge_solv1, edge_solv2], 0)
    src = jnp.concatenate([ei_solu[0], ei_solv1[0] + _N, ei_solv2[0] + 2 * _N])
    dst = jnp.concatenate([ei_solu[1], ei_solv1[1] + _N, ei_solv2[1] + 2 * _N])
    dst_r = dst.reshape(_NW, _NCH, _CH)
    bt = jnp.stack([bt_solu, bt_solv1, bt_solv2], 0)

    def stk(a, b):
        return jnp.stack([a, b, b], 0)

    pw3 = stk(gsolu_pW, gsolv_pW)
    pb3 = stk(gsolu_pb, gsolv_pb).reshape(_NT, 1, _D)
    e1w3 = stk(gsolu_e1W, gsolv_e1W)
    e1b3 = stk(gsolu_e1b, gsolv_e1b).reshape(_NT, 1, _EHID)
    e2w3 = stk(gsolu_e2W, gsolv_e2W)
    e2b3 = stk(gsolu_e2b, gsolv_e2b).reshape(_NT, 1, _D * _D)
    nnb3 = stk(gsolu_nnb, gsolv_nnb).reshape(_NT, 1, _D)
    wih3 = stk(gsolu_Wih, gsolv_Wih)
    whh3 = stk(gsolu_Whh, gsolv_Whh)
    bih3 = stk(gsolu_bih, gsolv_bih).reshape(_NT, 1, 3 * _D)
    bhh3 = stk(gsolu_bhh, gsolv_bhh).reshape(_NT, 1, 3 * _D)
    zero_rows = jnp.zeros((_RPS, _HP), jnp.float32)

    gather_rows, scatter_add = _sc_kernels()
    h, z = _proj(nf, ef, pw3, pb3, e1w3, e1b3)
    for _ in range(_NMP):
        hs = gather_rows(h, src)
        m = _message(z, hs, e2w3, e2b3)
        a2 = scatter_add(m, dst_r, zero_rows).reshape(_NC, _NTOT, _HP)
        h = _gru(a2, h, nnb3, wih3, whh3, bih3, bhh3)

    out = _head(h, bt,
                s2s_Wih0, s2s_Whh0, s2s_bih0.reshape(1, 4 * _D),
                s2s_bhh0.reshape(1, 4 * _D),
                s2s_Wih1, s2s_Whh1, s2s_bih1.reshape(1, 4 * _D),
                s2s_bhh1.reshape(1, 4 * _D),
                g_solv_facs, temp.reshape(_B, 1),
                fc1W[:4 * _D], fc1W[4 * _D:],
                fc1b.reshape(1, _D), fc2W, fc2b.reshape(1, 1))
    return out


# speedup vs baseline: 4.6615x; 2.0215x over previous
import functools

import jax
import jax.numpy as jnp
from jax import lax
from jax.experimental import pallas as pl
from jax.experimental.pallas import tpu as pltpu
from jax.experimental.pallas import tpu_sc as plsc

_N = 2560
_E = 5120
_B = 64
_D = 64
_EHID = 128
_NODE_IN = 128
_EDGE_IN = 16
_NT = 3
_NTOT = _NT * _N
_ETOT = _NT * _E
_NMP = 3
_NS2S = 3

_NC = 2
_NS = 16
_NW = _NC * _NS
_EPW = _ETOT // _NW
_CH = 120
_NCH = _EPW // _CH
_RPS = _NTOT // _NS

_HP = 128
_EB = 512
_NBLK = _E // _EB
_ICHUNK = 8


@functools.lru_cache(maxsize=None)
def _sc_kernels():
    mesh = plsc.VectorSubcoreMesh(core_axis_name="c", subcore_axis_name="s")

    @functools.partial(
        pl.kernel,
        mesh=mesh,
        out_type=jax.ShapeDtypeStruct((_ETOT, _HP), jnp.float32),
        scratch_types=[
            pltpu.VMEM((_CH,), jnp.int32),
            pltpu.VMEM((_CH, _HP), jnp.float32),
            pltpu.SemaphoreType.DMA,
        ],
    )
    def _gather_rows(h_hbm, src_hbm, out_hbm, idx_v, rows_v, sem):
        wid = lax.axis_index("c") * _NS + lax.axis_index("s")
        base = wid * _EPW
        for j in range(_NCH):
            off = base + j * _CH
            pltpu.sync_copy(src_hbm.at[pl.ds(off, _CH)], idx_v)
            pltpu.async_copy(h_hbm.at[idx_v], rows_v, sem).wait()
            pltpu.sync_copy(rows_v, out_hbm.at[pl.ds(off, _CH)])

    @functools.partial(
        pl.kernel,
        mesh=mesh,
        out_type=jax.ShapeDtypeStruct((_NC * _NTOT, _HP), jnp.float32),
        scratch_types=[
            pltpu.VMEM((_NCH, _CH), jnp.int32),
            pltpu.VMEM((_CH, _HP), jnp.float32),
            pltpu.VMEM_SHARED((_NTOT, _HP), jnp.float32),
        ],
    )
    def _scatter_add(m_hbm, dst_hbm, zero_hbm, out_hbm, idx_v, m_v, agg_sh):
        cid = lax.axis_index("c")
        sid = lax.axis_index("s")
        wid = cid * _NS + sid
        pltpu.sync_copy(zero_hbm, agg_sh.at[pl.ds(sid * _RPS, _RPS)])
        plsc.subcore_barrier()
        pltpu.sync_copy(dst_hbm.at[wid], idx_v)
        for j in range(_NCH):
            pltpu.sync_copy(m_hbm.at[pl.ds(wid * _EPW + j * _CH, _CH)], m_v)
            pltpu.sync_copy(m_v, agg_sh.at[idx_v.at[j]], add=True)
        plsc.subcore_barrier()
        pltpu.sync_copy(
            agg_sh.at[pl.ds(sid * _RPS, _RPS)],
            out_hbm.at[pl.ds(cid * _NTOT + sid * _RPS, _RPS)],
        )

    return _gather_rows, _scatter_add


def _proj_body(nf_ref, ef_ref, pw_ref, pb_ref, e1w_ref, e1b_ref, h_ref, z_ref):
    h = jnp.dot(nf_ref[...], pw_ref[0], preferred_element_type=jnp.float32)
    h = jnp.maximum(h + pb_ref[0], 0.0)
    h_ref[...] = jnp.concatenate([h, jnp.zeros((_N, _HP - _D), jnp.float32)], 1)
    z = jnp.dot(ef_ref[...], e1w_ref[0], preferred_element_type=jnp.float32)
    z_ref[...] = jnp.maximum(z + e1b_ref[0], 0.0)


_proj = pl.pallas_call(
    _proj_body,
    grid=(_NT,),
    in_specs=[
        pl.BlockSpec((_N, _NODE_IN), lambda t: (t, 0)),
        pl.BlockSpec((_E, _EDGE_IN), lambda t: (t, 0)),
        pl.BlockSpec((1, _NODE_IN, _D), lambda t: (t, 0, 0)),
        pl.BlockSpec((1, 1, _D), lambda t: (t, 0, 0)),
        pl.BlockSpec((1, _EDGE_IN, _EHID), lambda t: (t, 0, 0)),
        pl.BlockSpec((1, 1, _EHID), lambda t: (t, 0, 0)),
    ],
    out_specs=[
        pl.BlockSpec((_N, _HP), lambda t: (t, 0)),
        pl.BlockSpec((_E, _EHID), lambda t: (t, 0)),
    ],
    out_shape=[
        jax.ShapeDtypeStruct((_NTOT, _HP), jnp.float32),
        jax.ShapeDtypeStruct((_ETOT, _EHID), jnp.float32),
    ],
)


def _message_body(z_ref, hs_ref, w2_ref, b2_ref, m_ref):
    f32 = jnp.float32
    z = z_ref[...]
    hs = hs_ref[...]
    cw = _ICHUNK * _D
    rep = jnp.where(
        lax.broadcasted_iota(jnp.int32, (_ICHUNK, cw), 1) // _D
        == lax.broadcasted_iota(jnp.int32, (_ICHUNK, cw), 0), 1.0, 0.0
    ).astype(f32)
    fold = jnp.where(
        lax.broadcasted_iota(jnp.int32, (cw, _D), 0) % _D
        == lax.broadcasted_iota(jnp.int32, (cw, _D), 1), 1.0, 0.0
    ).astype(f32)
    acc = jnp.zeros((_EB, cw), f32)
    for i0 in range(0, _D, _ICHUNK):
        w2c = w2_ref[0][:, i0 * _D:(i0 + _ICHUNK) * _D]
        ewc = jnp.dot(z, w2c, preferred_element_type=f32)
        ewc = ewc + b2_ref[0][:, i0 * _D:(i0 + _ICHUNK) * _D]
        hrep = jnp.dot(hs[:, i0:i0 + _ICHUNK], rep, preferred_element_type=f32)
        acc = acc + hrep * ewc
    m = jnp.dot(acc, fold, preferred_element_type=f32)
    m_ref[...] = jnp.concatenate(
        [m, jnp.zeros((_EB, _HP - _D), f32)], 1)


_message = pl.pallas_call(
    _message_body,
    grid=(_NT, _NBLK),
    in_specs=[
        pl.BlockSpec((_EB, _EHID), lambda t, b: (t * _NBLK + b, 0)),
        pl.BlockSpec((_EB, _HP), lambda t, b: (t * _NBLK + b, 0)),
        pl.BlockSpec((1, _EHID, _D * _D), lambda t, b: (t, 0, 0)),
        pl.BlockSpec((1, 1, _D * _D), lambda t, b: (t, 0, 0)),
    ],
    out_specs=pl.BlockSpec((_EB, _HP), lambda t, b: (t * _NBLK + b, 0)),
    out_shape=jax.ShapeDtypeStruct((_ETOT, _HP), jnp.float32),
)


def _gru_body(a2_ref, h_ref, nnb_ref, wih_ref, whh_ref, bih_ref, bhh_ref,
              hout_ref):
    agg = a2_ref[0][:, :_D] + a2_ref[1][:, :_D] + nnb_ref[0]
    x = jnp.maximum(agg, 0.0)
    h = h_ref[...][:, :_D]
    gi = jnp.dot(x, wih_ref[0], preferred_element_type=jnp.float32) + bih_ref[0]
    gh = jnp.dot(h, whh_ref[0], preferred_element_type=jnp.float32) + bhh_ref[0]
    r = jax.nn.sigmoid(gi[:, :_D] + gh[:, :_D])
    z = jax.nn.sigmoid(gi[:, _D:2 * _D] + gh[:, _D:2 * _D])
    n = jnp.tanh(gi[:, 2 * _D:] + r * gh[:, 2 * _D:])
    hnew = (1.0 - z) * n + z * h
    hout_ref[...] = jnp.concatenate(
        [hnew, jnp.zeros((_N, _HP - _D), jnp.float32)], 1)


_gru = pl.pallas_call(
    _gru_body,
    grid=(_NT,),
    in_specs=[
        pl.BlockSpec((2, _N, _HP), lambda t: (0, t, 0)),
        pl.BlockSpec((_N, _HP), lambda t: (t, 0)),
        pl.BlockSpec((1, 1, _D), lambda t: (t, 0, 0)),
        pl.BlockSpec((1, _D, 3 * _D), lambda t: (t, 0, 0)),
        pl.BlockSpec((1, _D, 3 * _D), lambda t: (t, 0, 0)),
        pl.BlockSpec((1, 1, 3 * _D), lambda t: (t, 0, 0)),
        pl.BlockSpec((1, 1, 3 * _D), lambda t: (t, 0, 0)),
    ],
    out_specs=pl.BlockSpec((_N, _HP), lambda t: (t, 0)),
    out_shape=jax.ShapeDtypeStruct((_NTOT, _HP), jnp.float32),
)


def _head_body(h_ref, bt_ref, wih0_ref, whh0_ref, bih0_ref, bhh0_ref,
               wih1_ref, whh1_ref, bih1_ref, bhh1_ref, facs_ref, temp_ref,
               fc1wa_ref, fc1wb_ref, fc1b_ref, fc2w_ref, fc2b_ref, out_ref):
    f32 = jnp.float32
    h_all = h_ref[...]
    bt_all = bt_ref[...]
    iota_b = lax.broadcasted_iota(jnp.int32, (_B, _N), 0)
    q_stars = []
    for t in range(_NT):
        feat = h_all[t * _N:(t + 1) * _N, :_D]
        feat_t = feat.T
        seg = jnp.broadcast_to(bt_all[t:t + 1, :], (_B, _N))
        mask = seg == iota_b
        h0 = jnp.zeros((_B, _D), f32)
        c0 = jnp.zeros((_B, _D), f32)
        h1 = jnp.zeros((_B, _D), f32)
        c1 = jnp.zeros((_B, _D), f32)
        q_star = jnp.zeros((_B, 2 * _D), f32)
        for _ in range(_NS2S):
            g0 = (jnp.dot(q_star, wih0_ref[...], preferred_element_type=f32)
                  + bih0_ref[0]
                  + jnp.dot(h0, whh0_ref[...], preferred_element_type=f32)
                  + bhh0_ref[0])
            c0 = (jax.nn.sigmoid(g0[:, _D:2 * _D]) * c0
                  + jax.nn.sigmoid(g0[:, :_D]) * jnp.tanh(g0[:, 2 * _D:3 * _D]))
            h0 = jax.nn.sigmoid(g0[:, 3 * _D:]) * jnp.tanh(c0)
            g1 = (jnp.dot(h0, wih1_ref[...], preferred_element_type=f32)
                  + bih1_ref[0]
                  + jnp.dot(h1, whh1_ref[...], preferred_element_type=f32)
                  + bhh1_ref[0])
            c1 = (jax.nn.sigmoid(g1[:, _D:2 * _D]) * c1
                  + jax.nn.sigmoid(g1[:, :_D]) * jnp.tanh(g1[:, 2 * _D:3 * _D]))
            h1 = jax.nn.sigmoid(g1[:, 3 * _D:]) * jnp.tanh(c1)
            q = h1
            s = jnp.dot(q, feat_t, preferred_element_type=f32)
            smask = jnp.where(mask, s, -jnp.inf)
            emax = jnp.max(smask, axis=1, keepdims=True)
            emax = jnp.where(emax > -3e38, emax, 0.0)
            a = jnp.where(mask, jnp.exp(s - emax), 0.0)
            denom = jnp.sum(a, axis=1, keepdims=True)
            denom = jnp.where(denom == 0.0, 1.0, denom)
            readout = jnp.dot(a / denom, feat, preferred_element_type=f32)
            q_star = jnp.concatenate([q, readout], axis=1)
        q_stars.append(q_star)
    facs = facs_ref[...]
    gb = facs[:, 0:1] * q_stars[1] + facs[:, 1:2] * q_stars[2]
    tnorm = (temp_ref[...] - 30.0) / 15.0
    x = jnp.concatenate([q_stars[0], gb], axis=1)
    hid = (jnp.dot(x, fc1wa_ref[...], preferred_element_type=f32)
           + tnorm * fc1wb_ref[...] + fc1b_ref[...])
    hid = jnp.maximum(hid, 0.0)
    out_ref[...] = (jnp.dot(hid, fc2w_ref[...], preferred_element_type=f32)
                    + fc2b_ref[...])


_head = pl.pallas_call(
    _head_body,
    out_shape=jax.ShapeDtypeStruct((_B, 1), jnp.float32),
)


def kernel(node_solu, edge_solu, ei_solu, bt_solu,
           node_solv1, edge_solv1, ei_solv1, bt_solv1,
           node_solv2, edge_solv2, ei_solv2, bt_solv2,
           g_solv_facs, temp,
           gsolu_pW, gsolu_pb, gsolu_e1W, gsolu_e1b, gsolu_e2W, gsolu_e2b,
           gsolu_nnb, gsolu_Wih, gsolu_Whh, gsolu_bih, gsolu_bhh,
           gsolv_pW, gsolv_pb, gsolv_e1W, gsolv_e1b, gsolv_e2W, gsolv_e2b,
           gsolv_nnb, gsolv_Wih, gsolv_Whh, gsolv_bih, gsolv_bhh,
           s2s_Wih0, s2s_Whh0, s2s_bih0, s2s_bhh0,
           s2s_Wih1, s2s_Whh1, s2s_bih1, s2s_bhh1,
           fc1W, fc1b, fc2W, fc2b):
    nf = jnp.concatenate([node_solu, node_solv1, node_solv2], 0)
    ef = jnp.concatenate([edge_solu, edge_solv1, edge_solv2], 0)
    src = jnp.concatenate([ei_solu[0], ei_solv1[0] + _N, ei_solv2[0] + 2 * _N])
    dst = jnp.concatenate([ei_solu[1], ei_solv1[1] + _N, ei_solv2[1] + 2 * _N])
    dst_r = dst.reshape(_NW, _NCH, _CH)
    bt = jnp.stack([bt_solu, bt_solv1, bt_solv2], 0)

    def stk(a, b):
        return jnp.stack([a, b, b], 0)

    pw3 = stk(gsolu_pW, gsolv_pW)
    pb3 = stk(gsolu_pb, gsolv_pb).reshape(_NT, 1, _D)
    e1w3 = stk(gsolu_e1W, gsolv_e1W)
    e1b3 = stk(gsolu_e1b, gsolv_e1b).reshape(_NT, 1, _EHID)
    e2w3 = stk(gsolu_e2W, gsolv_e2W)
    e2b3 = stk(gsolu_e2b, gsolv_e2b).reshape(_NT, 1, _D * _D)
    nnb3 = stk(gsolu_nnb, gsolv_nnb).reshape(_NT, 1, _D)
    wih3 = stk(gsolu_Wih, gsolv_Wih)
    whh3 = stk(gsolu_Whh, gsolv_Whh)
    bih3 = stk(gsolu_bih, gsolv_bih).reshape(_NT, 1, 3 * _D)
    bhh3 = stk(gsolu_bhh, gsolv_bhh).reshape(_NT, 1, 3 * _D)
    zero_rows = jnp.zeros((_RPS, _HP), jnp.float32)

    gather_rows, scatter_add = _sc_kernels()
    h, z = _proj(nf, ef, pw3, pb3, e1w3, e1b3)
    for _ in range(_NMP):
        hs = gather_rows(h, src)
        m = _message(z, hs, e2w3, e2b3)
        a2 = scatter_add(m, dst_r, zero_rows).reshape(_NC, _NTOT, _HP)
        h = _gru(a2, h, nnb3, wih3, whh3, bih3, bhh3)

    out = _head(h, bt,
                s2s_Wih0, s2s_Whh0, s2s_bih0.reshape(1, 4 * _D),
                s2s_bhh0.reshape(1, 4 * _D),
                s2s_Wih1, s2s_Whh1, s2s_bih1.reshape(1, 4 * _D),
                s2s_bhh1.reshape(1, 4 * _D),
                g_solv_facs, temp.reshape(_B, 1),
                fc1W[:4 * _D], fc1W[4 * _D:],
                fc1b.reshape(1, _D), fc2W, fc2b.reshape(1, 1))
    return out
